# baseline (speedup 1.0000x reference)
import dataclasses
import functools

import jax
import jax.numpy as jnp
from jax import lax
from jax.experimental import pallas as pl
from jax.experimental.pallas import tpu as pltpu
from jax.experimental.pallas import tpu_sc as plsc

B = 4096
C = 6
E = 128
L = 16
NC = 2
NS = 16
NW = NC * NS
BPW = B // NW
NCH = 4
RPC = BPW // NCH
GR = 16


def _dots_sc(target, context, target_table, context_table):
  mesh = plsc.VectorSubcoreMesh(core_axis_name="c", subcore_axis_name="s")

  cp = pltpu.CompilerParams()
  if "needs_layout_passes" in pltpu.CompilerParams.__dataclass_fields__:
    cp = dataclasses.replace(cp, needs_layout_passes=False)

  @functools.partial(
      pl.kernel,
      compiler_params=cp,
      out_type=jax.ShapeDtypeStruct((B, C), jnp.float32),
      mesh=mesh,
      scratch_types=[
          pltpu.VMEM((BPW,), jnp.int32),
          pltpu.VMEM((BPW, C), jnp.int32),
          pltpu.VMEM((2, RPC, E), jnp.float32),
          pltpu.VMEM((2, C, RPC, E), jnp.float32),
          pltpu.VMEM((BPW, C), jnp.float32),
          pltpu.SemaphoreType.DMA,
          pltpu.SemaphoreType.DMA,
      ],
  )
  def k(tgt_hbm, ctx_hbm, ttab_hbm, ctab_hbm, out_hbm,
        tidx_v, cidx_v, trows_v, crows_v, out_v, sem0, sem1):
    wid = lax.axis_index("c") * NS + lax.axis_index("s")
    base = wid * BPW
    sems = (sem0, sem1)

    pltpu.sync_copy(tgt_hbm.at[pl.ds(base, BPW)], tidx_v)
    pltpu.sync_copy(ctx_hbm.at[pl.ds(base, BPW)], cidx_v)

    lanes = jax.lax.iota(jnp.int32, L)
    lane_mask = lanes < C

    def fire(g, p):
      cps = [
          pltpu.async_copy(
              ttab_hbm.at[tidx_v.at[pl.ds(g * RPC, RPC)]],
              trows_v.at[p], sems[p]),
      ]
      for c in range(C):
        for h in range(RPC // GR):
          rows = g * RPC + h * GR + lanes
          cols = jnp.full((L,), c, jnp.int32)
          idx = plsc.load_gather(cidx_v, [rows, cols])
          cps.append(
              pltpu.async_copy(
                  ctab_hbm.at[idx],
                  crows_v.at[p].at[c].at[pl.ds(h * GR, GR)],
                  sems[p]))
      return cps

    inflight = fire(0, 0)
    for g in range(NCH):
      p = g % 2
      cur = inflight
      if g + 1 < NCH:
        inflight = fire(g + 1, (g + 1) % 2)
      for cpy in cur:
        cpy.wait()

      tbuf = trows_v.at[p]
      cbuf = crows_v.at[p]

      @pl.loop(0, RPC)
      def _(i):
        t = [tbuf[i, pl.ds(kk * L, L)] for kk in range(E // L)]
        res = jnp.zeros((L,), jnp.float32)
        for c in range(C):
          acc = t[0] * cbuf[c, i, pl.ds(0, L)]
          for kk in range(1, E // L):
            acc = acc + t[kk] * cbuf[c, i, pl.ds(kk * L, L)]
          res = jnp.where(lanes == c, jnp.sum(acc), res)
        row = jnp.full((L,), g * RPC, jnp.int32) + i
        plsc.store_scatter(out_v, [row, lanes], res, mask=lane_mask)

    pltpu.sync_copy(out_v, out_hbm.at[pl.ds(base, BPW)])

  return k(target, context, target_table, context_table)


def kernel(target, context, target_table, context_table):
  return _dots_sc(target, context, target_table, context_table)


# speedup vs baseline: 1.0158x; 1.0158x over previous
import dataclasses
import functools

import jax
import jax.numpy as jnp
from jax import lax
from jax.experimental import pallas as pl
from jax.experimental.pallas import tpu as pltpu
from jax.experimental.pallas import tpu_sc as plsc

B = 4096
C = 6
E = 128
L = 16
NC = 2
NS = 16
NW = NC * NS
BPW = B // NW
NCH = 4
RPC = BPW // NCH
GR = 16


def _dots_sc(target, context, target_table, context_table):
  mesh = plsc.VectorSubcoreMesh(core_axis_name="c", subcore_axis_name="s")

  cp = pltpu.CompilerParams()
  if "needs_layout_passes" in pltpu.CompilerParams.__dataclass_fields__:
    cp = dataclasses.replace(cp, needs_layout_passes=False)

  @functools.partial(
      pl.kernel,
      compiler_params=cp,
      out_type=jax.ShapeDtypeStruct((B, C), jnp.float32),
      mesh=mesh,
      scratch_types=[
          pltpu.VMEM((BPW,), jnp.int32),
          pltpu.VMEM((BPW, C), jnp.int32),
          pltpu.VMEM((BPW * C,), jnp.int32),
          pltpu.VMEM((2, RPC, E), jnp.float32),
          pltpu.VMEM((2, RPC * C, E), jnp.float32),
          pltpu.VMEM((BPW, C), jnp.float32),
          pltpu.SemaphoreType.DMA,
          pltpu.SemaphoreType.DMA,
      ],
  )
  def k(tgt_hbm, ctx_hbm, ttab_hbm, ctab_hbm, out_hbm,
        tidx_v, cidx_v, cflat_v, trows_v, crows_v, out_v, sem0, sem1):
    wid = lax.axis_index("c") * NS + lax.axis_index("s")
    base = wid * BPW
    sems = (sem0, sem1)

    pltpu.sync_copy(tgt_hbm.at[pl.ds(base, BPW)], tidx_v)
    pltpu.sync_copy(ctx_hbm.at[pl.ds(base, BPW)], cidx_v)

    lanes = jax.lax.iota(jnp.int32, L)
    lane_mask = lanes < C
    CPC = RPC * C

    def fire(g, p):
      for h in range(RPC // GR):
        rows = g * RPC + h * GR + lanes
        for c in range(C):
          vals = plsc.load_gather(cidx_v, [rows, lanes * 0 + c])
          plsc.store_scatter(cflat_v, [rows * C + c], vals)
      cps = [
          pltpu.async_copy(
              ttab_hbm.at[tidx_v.at[pl.ds(g * RPC, RPC)]],
              trows_v.at[p], sems[p]),
      ]
      for h in range(2):
        cps.append(
            pltpu.async_copy(
                ctab_hbm.at[cflat_v.at[pl.ds(g * CPC + h * CPC // 2, CPC // 2)]],
                crows_v.at[p].at[pl.ds(h * CPC // 2, CPC // 2)],
                sems[p]))
      return cps

    inflight = fire(0, 0)
    for g in range(NCH):
      p = g % 2
      cur = inflight
      if g + 1 < NCH:
        inflight = fire(g + 1, (g + 1) % 2)
      for cpy in cur:
        cpy.wait()

      tbuf = trows_v.at[p]
      cbuf = crows_v.at[p]

      @plsc.parallel_loop(0, RPC, unroll=2)
      def _(i):
        t = [tbuf[i, pl.ds(kk * L, L)] for kk in range(E // L)]
        res = jnp.zeros((L,), jnp.float32)
        for c in range(C):
          acc0 = t[0] * cbuf[i * C + c, pl.ds(0, L)]
          acc1 = t[1] * cbuf[i * C + c, pl.ds(L, L)]
          for kk in range(2, E // L, 2):
            acc0 = acc0 + t[kk] * cbuf[i * C + c, pl.ds(kk * L, L)]
            acc1 = acc1 + t[kk + 1] * cbuf[i * C + c, pl.ds((kk + 1) * L, L)]
          res = jnp.where(lanes == c, jnp.sum(acc0 + acc1), res)
        row = jnp.full((L,), g * RPC, jnp.int32) + i
        plsc.store_scatter(out_v, [row, lanes], res, mask=lane_mask)

    pltpu.sync_copy(out_v, out_hbm.at[pl.ds(base, BPW)])

  return k(target, context, target_table, context_table)


def kernel(target, context, target_table, context_table):
  return _dots_sc(target, context, target_table, context_table)


# speedup vs baseline: 1.0684x; 1.0518x over previous
import dataclasses
import functools

import jax
import jax.numpy as jnp
from jax import lax
from jax.experimental import pallas as pl
from jax.experimental.pallas import tpu as pltpu
from jax.experimental.pallas import tpu_sc as plsc

B = 4096
C = 6
E = 128
L = 16
NC = 2
NS = 16
NW = NC * NS
BPW = B // NW
NCH = 4
RPC = BPW // NCH
CPC = RPC * C
HALF = CPC // 2


def _dots_sc(target, ctx_flat, target_table, context_table):
  mesh = plsc.VectorSubcoreMesh(core_axis_name="c", subcore_axis_name="s")

  cp = pltpu.CompilerParams()
  if "needs_layout_passes" in pltpu.CompilerParams.__dataclass_fields__:
    cp = dataclasses.replace(cp, needs_layout_passes=False)

  @functools.partial(
      pl.kernel,
      compiler_params=cp,
      out_type=jax.ShapeDtypeStruct((C, B), jnp.float32),
      mesh=mesh,
      scratch_types=[
          pltpu.VMEM((BPW,), jnp.int32),
          pltpu.VMEM((BPW * C,), jnp.int32),
          pltpu.VMEM((2, RPC, E), jnp.float32),
          pltpu.VMEM((2, C, RPC, E), jnp.float32),
          pltpu.VMEM((C, BPW), jnp.float32),
          pltpu.SemaphoreType.DMA,
          pltpu.SemaphoreType.DMA,
      ],
  )
  def k(tgt_hbm, ctx_hbm, ttab_hbm, ctab_hbm, out_hbm,
        tidx_v, cidx_v, trows_v, crows_v, out_v, sem0, sem1):
    wid = lax.axis_index("c") * NS + lax.axis_index("s")
    base = wid * BPW
    sems = (sem0, sem1)

    pltpu.sync_copy(tgt_hbm.at[pl.ds(base, BPW)], tidx_v)
    for c in range(C):
      pltpu.sync_copy(ctx_hbm.at[c].at[pl.ds(base, BPW)],
                      cidx_v.at[pl.ds(c * BPW, BPW)])

    def fire(g, p):
      return [
          pltpu.async_copy(
              ttab_hbm.at[tidx_v.at[pl.ds(g * RPC, RPC)]],
              trows_v.at[p], sems[p]),
      ] + [
          pltpu.async_copy(
              ctab_hbm.at[cidx_v.at[pl.ds(c * BPW + g * RPC, RPC)]],
              crows_v.at[p].at[c], sems[p])
          for c in range(C)
      ]

    lanes = jax.lax.iota(jnp.int32, L)
    lane_mask = lanes < C

    inflight = fire(0, 0)
    for g in range(NCH):
      p = g % 2
      cur = inflight
      if g + 1 < NCH:
        inflight = fire(g + 1, (g + 1) % 2)
      for cpy in cur:
        cpy.wait()

      tbuf = trows_v.at[p]
      cbuf = crows_v.at[p]

      @pl.loop(0, RPC)
      def _(i):
        t = [tbuf[i, pl.ds(kk * L, L)] for kk in range(E // L)]
        res = jnp.zeros((L,), jnp.float32)
        for c in range(C):
          acc = t[0] * cbuf[c, i, pl.ds(0, L)]
          for kk in range(1, E // L):
            acc = acc + t[kk] * cbuf[c, i, pl.ds(kk * L, L)]
          res = jnp.where(lanes == c, jnp.sum(acc), res)
        orow = jnp.full((L,), g * RPC, jnp.int32) + i
        plsc.store_scatter(out_v, [lanes, orow], res, mask=lane_mask)

    pltpu.sync_copy(out_v, out_hbm.at[pl.ds(0, C), pl.ds(base, BPW)])

  return k(target, ctx_flat, target_table, context_table)


def kernel(target, context, target_table, context_table):
  return _dots_sc(target, context.T, target_table, context_table).T
